# baseline (speedup 1.0000x reference)
import functools

import jax
import jax.numpy as jnp
from jax.experimental import pallas as pl
from jax.experimental.pallas import tpu as pltpu


_K = 8


def _dense_stages(x_ref, w1_ref, b1_ref, w2_ref, b2_ref):
    z = jax.lax.dot_general(x_ref[...], w1_ref[...],
                            (((1,), (1,)), ((), ())),
                            preferred_element_type=jnp.float32)
    h = jax.nn.silu(z + b1_ref[...])
    logits = jax.lax.dot_general(h, w2_ref[...],
                                 (((1,), (1,)), ((), ())),
                                 preferred_element_type=jnp.float32)
    return jax.nn.sigmoid(logits + b2_ref[...])


def _fast_body(x_ref, w1_ref, b1_ref, w2_ref, b2_ref, ema_ref,
               coeffs_ref, mon_ref, cv_ref, cnt_ref,
               *, n_blocks, n_experts):
    i = pl.program_id(0)

    z = jax.lax.dot_general(x_ref[...], w1_ref[...],
                            (((1,), (1,)), ((), ())),
                            preferred_element_type=jnp.float32)
    h = jax.nn.silu(z + b1_ref[...])
    logits_t = jax.lax.dot_general(w2_ref[...], h,
                                   (((1,), (1,)), ((), ())),
                                   preferred_element_type=jnp.float32)
    scores_t = jax.nn.sigmoid(logits_t + b2_ref[...])

    bn = coeffs_ref.shape[0]
    chunk = 128
    cnts = []
    parts = []
    for c in range(0, bn, chunk):
        sc = jax.lax.slice(scores_t, (0, c), (n_experts, c + chunk))
        masked = sc
        denom = jnp.full((1, chunk), 1e-8, jnp.float32)
        rowmax = None
        for k in range(_K):
            m = jnp.max(masked, axis=0, keepdims=True)
            masked = jnp.where(masked == m, -1.0, masked)
            denom = denom + m
            if k == 0:
                rowmax = m
        sel = masked == -1.0
        recip = 1.0 / denom
        coeffs_t = jnp.where(sel, sc, 0.0) * recip
        coeffs_ref[pl.ds(c, chunk), :] = coeffs_t.T
        cnts.append(jnp.sum(jnp.where(sel, 1.0, 0.0)))
        parts.append(jnp.sum(rowmax * recip))

    @pl.when(i == 0)
    def _init():
        mon_ref[0, 0] = 0.0
        cnt_ref[0, 0] = 0.0
        e = ema_ref[...]
        mu = jnp.sum(e) / n_experts
        var = jnp.sum((e - mu) ** 2) / (n_experts - 1)
        cv_ref[0, 0] = jnp.sqrt(var) / (mu + 1e-8)

    mon_ref[0, 0] = mon_ref[0, 0] + sum(parts)
    cnt_ref[0, 0] = cnt_ref[0, 0] + sum(cnts)


def _exact_body(x_ref, w1_ref, b1_ref, w2_ref, b2_ref,
                coeffs_ref, mon_ref, *, n_blocks, n_rows, n_experts):
    i = pl.program_id(0)
    scores = _dense_stages(x_ref, w1_ref, b1_ref, w2_ref, b2_ref)

    iota = jax.lax.broadcasted_iota(jnp.int32, scores.shape, 1)
    masked = scores
    sel = jnp.zeros(scores.shape, jnp.bool_)
    for _ in range(_K):
        m = jnp.max(masked, axis=1, keepdims=True)
        elig = masked == m
        fidx = jnp.min(jnp.where(elig, iota, n_experts), axis=1,
                       keepdims=True)
        first = iota == fidx
        sel = jnp.logical_or(sel, first)
        masked = jnp.where(first, -1.0, masked)
    selscores = jnp.where(sel, scores, 0.0)
    denom = jnp.sum(selscores, axis=1, keepdims=True) + 1e-8
    coeffs_ref[...] = selscores / denom
    part = jnp.sum(jnp.max(scores, axis=1, keepdims=True) / denom)

    @pl.when(i == 0)
    def _init():
        mon_ref[0, 0] = 0.0

    mon_ref[0, 0] = mon_ref[0, 0] + part

    @pl.when(i == n_blocks - 1)
    def _final():
        mon_ref[0, 0] = mon_ref[0, 0] / n_rows


def kernel(global_features, W1, b1, W2, b2, ema_load):
    n, d = global_features.shape
    h_dim = W1.shape[0]
    e_dim = W2.shape[0]
    bn = 1024
    n_blocks = n // bn

    b1r = b1.reshape(1, h_dim)
    b2r = b2.reshape(1, e_dim)

    x_spec = pl.BlockSpec((bn, d), lambda i: (i, 0))
    w1_spec = pl.BlockSpec((h_dim, d), lambda i: (0, 0))
    b1_spec = pl.BlockSpec((1, h_dim), lambda i: (0, 0))
    w2_spec = pl.BlockSpec((e_dim, h_dim), lambda i: (0, 0))
    b2_spec = pl.BlockSpec((1, e_dim), lambda i: (0, 0))
    smem_spec = pl.BlockSpec((1, 1), lambda i: (0, 0),
                             memory_space=pltpu.SMEM)

    fast = functools.partial(_fast_body, n_blocks=n_blocks,
                             n_experts=e_dim)
    coeffs_f, mon_f, cv, cnt = pl.pallas_call(
        fast,
        grid=(n_blocks,),
        in_specs=[x_spec, w1_spec, b1_spec, w2_spec,
                  pl.BlockSpec((e_dim, 1), lambda i: (0, 0)),
                  pl.BlockSpec((1, e_dim), lambda i: (0, 0))],
        out_specs=[pl.BlockSpec((bn, e_dim), lambda i: (i, 0)),
                   smem_spec, smem_spec, smem_spec],
        out_shape=[
            jax.ShapeDtypeStruct((n, e_dim), jnp.float32),
            jax.ShapeDtypeStruct((1, 1), jnp.float32),
            jax.ShapeDtypeStruct((1, 1), jnp.float32),
            jax.ShapeDtypeStruct((1, 1), jnp.float32),
        ],
    )(global_features, W1, b1r, W2, b2.reshape(e_dim, 1),
      ema_load.reshape(1, e_dim))

    def _fast_result():
        return coeffs_f, mon_f[0, 0] / n

    def _exact_result():
        exact = functools.partial(_exact_body, n_blocks=n_blocks,
                                  n_rows=n, n_experts=e_dim)
        coeffs_e, mon_e = pl.pallas_call(
            exact,
            grid=(n_blocks,),
            in_specs=[x_spec, w1_spec, b1_spec, w2_spec, b2_spec],
            out_specs=[pl.BlockSpec((bn, e_dim), lambda i: (i, 0)),
                       smem_spec],
            out_shape=[
                jax.ShapeDtypeStruct((n, e_dim), jnp.float32),
                jax.ShapeDtypeStruct((1, 1), jnp.float32),
            ],
        )(global_features, W1, b1r, W2, b2r)
        return coeffs_e, mon_e[0, 0]

    coeffs, mon = jax.lax.cond(cnt[0, 0] == float(_K * n),
                               _fast_result, _exact_result)
    return coeffs, mon, cv[0, 0]


# speedup vs baseline: 1.0032x; 1.0032x over previous
import functools

import jax
import jax.numpy as jnp
from jax.experimental import pallas as pl
from jax.experimental.pallas import tpu as pltpu


_K = 8


def _dense_stages(x_ref, w1_ref, b1_ref, w2_ref, b2_ref):
    z = jax.lax.dot_general(x_ref[...], w1_ref[...],
                            (((1,), (1,)), ((), ())),
                            preferred_element_type=jnp.float32)
    h = jax.nn.silu(z + b1_ref[...])
    logits = jax.lax.dot_general(h, w2_ref[...],
                                 (((1,), (1,)), ((), ())),
                                 preferred_element_type=jnp.float32)
    return jax.nn.sigmoid(logits + b2_ref[...])


def _fast_body(x_ref, w1_ref, b1_ref, w2_ref, b2_ref, ema_ref,
               coeffs_ref, mon_ref, cv_ref, cnt_ref,
               *, n_blocks, n_experts):
    i = pl.program_id(0)

    z = jax.lax.dot_general(x_ref[...], w1_ref[...],
                            (((1,), (1,)), ((), ())),
                            preferred_element_type=jnp.float32)
    h = jax.nn.silu(z + b1_ref[...])
    logits_t = jax.lax.dot_general(w2_ref[...], h,
                                   (((1,), (1,)), ((), ())),
                                   preferred_element_type=jnp.float32)
    scores_t = jax.nn.sigmoid(logits_t + b2_ref[...])

    bn = coeffs_ref.shape[0]
    chunk = 256
    cnts = []
    parts = []
    for c in range(0, bn, chunk):
        sc = jax.lax.slice(scores_t, (0, c), (n_experts, c + chunk))
        masked = sc
        denom = jnp.full((1, chunk), 1e-8, jnp.float32)
        rowmax = None
        for k in range(_K):
            m = jnp.max(masked, axis=0, keepdims=True)
            masked = jnp.where(masked == m, -1.0, masked)
            denom = denom + m
            if k == 0:
                rowmax = m
        sel = masked == -1.0
        recip = 1.0 / denom
        coeffs_t = jnp.where(sel, sc, 0.0) * recip
        coeffs_ref[pl.ds(c, chunk), :] = coeffs_t.T
        cnts.append(jnp.sum(jnp.where(sel, 1.0, 0.0)))
        parts.append(jnp.sum(rowmax * recip))

    @pl.when(i == 0)
    def _init():
        mon_ref[0, 0] = 0.0
        cnt_ref[0, 0] = 0.0
        e = ema_ref[...]
        mu = jnp.sum(e) / n_experts
        var = jnp.sum((e - mu) ** 2) / (n_experts - 1)
        cv_ref[0, 0] = jnp.sqrt(var) / (mu + 1e-8)

    mon_ref[0, 0] = mon_ref[0, 0] + sum(parts)
    cnt_ref[0, 0] = cnt_ref[0, 0] + sum(cnts)


def _exact_body(x_ref, w1_ref, b1_ref, w2_ref, b2_ref,
                coeffs_ref, mon_ref, *, n_blocks, n_rows, n_experts):
    i = pl.program_id(0)
    scores = _dense_stages(x_ref, w1_ref, b1_ref, w2_ref, b2_ref)

    iota = jax.lax.broadcasted_iota(jnp.int32, scores.shape, 1)
    masked = scores
    sel = jnp.zeros(scores.shape, jnp.bool_)
    for _ in range(_K):
        m = jnp.max(masked, axis=1, keepdims=True)
        elig = masked == m
        fidx = jnp.min(jnp.where(elig, iota, n_experts), axis=1,
                       keepdims=True)
        first = iota == fidx
        sel = jnp.logical_or(sel, first)
        masked = jnp.where(first, -1.0, masked)
    selscores = jnp.where(sel, scores, 0.0)
    denom = jnp.sum(selscores, axis=1, keepdims=True) + 1e-8
    coeffs_ref[...] = selscores / denom
    part = jnp.sum(jnp.max(scores, axis=1, keepdims=True) / denom)

    @pl.when(i == 0)
    def _init():
        mon_ref[0, 0] = 0.0

    mon_ref[0, 0] = mon_ref[0, 0] + part

    @pl.when(i == n_blocks - 1)
    def _final():
        mon_ref[0, 0] = mon_ref[0, 0] / n_rows


def kernel(global_features, W1, b1, W2, b2, ema_load):
    n, d = global_features.shape
    h_dim = W1.shape[0]
    e_dim = W2.shape[0]
    bn = 1024
    n_blocks = n // bn

    b1r = b1.reshape(1, h_dim)
    b2r = b2.reshape(1, e_dim)

    x_spec = pl.BlockSpec((bn, d), lambda i: (i, 0))
    w1_spec = pl.BlockSpec((h_dim, d), lambda i: (0, 0))
    b1_spec = pl.BlockSpec((1, h_dim), lambda i: (0, 0))
    w2_spec = pl.BlockSpec((e_dim, h_dim), lambda i: (0, 0))
    b2_spec = pl.BlockSpec((1, e_dim), lambda i: (0, 0))
    smem_spec = pl.BlockSpec((1, 1), lambda i: (0, 0),
                             memory_space=pltpu.SMEM)

    fast = functools.partial(_fast_body, n_blocks=n_blocks,
                             n_experts=e_dim)
    coeffs_f, mon_f, cv, cnt = pl.pallas_call(
        fast,
        grid=(n_blocks,),
        in_specs=[x_spec, w1_spec, b1_spec, w2_spec,
                  pl.BlockSpec((e_dim, 1), lambda i: (0, 0)),
                  pl.BlockSpec((1, e_dim), lambda i: (0, 0))],
        out_specs=[pl.BlockSpec((bn, e_dim), lambda i: (i, 0)),
                   smem_spec, smem_spec, smem_spec],
        out_shape=[
            jax.ShapeDtypeStruct((n, e_dim), jnp.float32),
            jax.ShapeDtypeStruct((1, 1), jnp.float32),
            jax.ShapeDtypeStruct((1, 1), jnp.float32),
            jax.ShapeDtypeStruct((1, 1), jnp.float32),
        ],
    )(global_features, W1, b1r, W2, b2.reshape(e_dim, 1),
      ema_load.reshape(1, e_dim))

    def _fast_result():
        return coeffs_f, mon_f[0, 0] / n

    def _exact_result():
        exact = functools.partial(_exact_body, n_blocks=n_blocks,
                                  n_rows=n, n_experts=e_dim)
        coeffs_e, mon_e = pl.pallas_call(
            exact,
            grid=(n_blocks,),
            in_specs=[x_spec, w1_spec, b1_spec, w2_spec, b2_spec],
            out_specs=[pl.BlockSpec((bn, e_dim), lambda i: (i, 0)),
                       smem_spec],
            out_shape=[
                jax.ShapeDtypeStruct((n, e_dim), jnp.float32),
                jax.ShapeDtypeStruct((1, 1), jnp.float32),
            ],
        )(global_features, W1, b1r, W2, b2r)
        return coeffs_e, mon_e[0, 0]

    coeffs, mon = jax.lax.cond(cnt[0, 0] == float(_K * n),
                               _fast_result, _exact_result)
    return coeffs, mon, cv[0, 0]


# speedup vs baseline: 1.0038x; 1.0006x over previous
import functools

import jax
import jax.numpy as jnp
from jax.experimental import pallas as pl
from jax.experimental.pallas import tpu as pltpu


_K = 8


def _dense_stages(x_ref, w1_ref, b1_ref, w2_ref, b2_ref):
    z = jax.lax.dot_general(x_ref[...], w1_ref[...],
                            (((1,), (1,)), ((), ())),
                            preferred_element_type=jnp.float32)
    h = jax.nn.silu(z + b1_ref[...])
    logits = jax.lax.dot_general(h, w2_ref[...],
                                 (((1,), (1,)), ((), ())),
                                 preferred_element_type=jnp.float32)
    return jax.nn.sigmoid(logits + b2_ref[...])


def _fast_body(x_ref, w1_ref, b1_ref, w2_ref, b2_ref, ema_ref,
               coeffs_ref, mon_ref, cv_ref, cnt_ref, *, n_experts):
    i = pl.program_id(0)

    z = jax.lax.dot_general(x_ref[...], w1_ref[...],
                            (((1,), (1,)), ((), ())),
                            preferred_element_type=jnp.float32)
    h = jax.nn.silu(z + b1_ref[...])
    logits_t = jax.lax.dot_general(w2_ref[...], h,
                                   (((1,), (1,)), ((), ())),
                                   preferred_element_type=jnp.float32)
    scores_t = jax.nn.sigmoid(logits_t + b2_ref[...])

    bn = coeffs_ref.shape[0]
    chunk = 256
    cnts = []
    parts = []
    for c in range(0, bn, chunk):
        sc = jax.lax.slice(scores_t, (0, c), (n_experts, c + chunk))
        masked = sc
        denom = jnp.full((1, chunk), 1e-8, jnp.float32)
        rowmax = None
        for k in range(_K):
            m = jnp.max(masked, axis=0, keepdims=True)
            masked = jnp.where(masked == m, -1.0, masked)
            denom = denom + m
            if k == 0:
                rowmax = m
        sel = masked == -1.0
        recip = 1.0 / denom
        coeffs_t = jnp.where(sel, sc, 0.0) * recip
        coeffs_ref[pl.ds(c, chunk), :] = coeffs_t.T
        cnts.append(jnp.sum(jnp.where(sel, 1.0, 0.0)))
        parts.append(jnp.sum(rowmax * recip))

    @pl.when(i == 0)
    def _init():
        mon_ref[0, 0] = 0.0
        cnt_ref[0, 0] = 0.0
        e = ema_ref[...]
        mu = jnp.sum(e) / n_experts
        var = jnp.sum((e - mu) ** 2) / (n_experts - 1)
        cv_ref[0, 0] = jnp.sqrt(var) / (mu + 1e-8)

    mon_ref[0, 0] = mon_ref[0, 0] + sum(parts)
    cnt_ref[0, 0] = cnt_ref[0, 0] + sum(cnts)


def _exact_body(x_ref, w1_ref, b1_ref, w2_ref, b2_ref,
                coeffs_ref, mon_ref, *, n_blocks, n_rows, n_experts):
    i = pl.program_id(0)
    scores = _dense_stages(x_ref, w1_ref, b1_ref, w2_ref, b2_ref)

    iota = jax.lax.broadcasted_iota(jnp.int32, scores.shape, 1)
    masked = scores
    sel = jnp.zeros(scores.shape, jnp.bool_)
    for _ in range(_K):
        m = jnp.max(masked, axis=1, keepdims=True)
        elig = masked == m
        fidx = jnp.min(jnp.where(elig, iota, n_experts), axis=1,
                       keepdims=True)
        first = iota == fidx
        sel = jnp.logical_or(sel, first)
        masked = jnp.where(first, -1.0, masked)
    selscores = jnp.where(sel, scores, 0.0)
    denom = jnp.sum(selscores, axis=1, keepdims=True) + 1e-8
    coeffs_ref[...] = selscores / denom
    part = jnp.sum(jnp.max(scores, axis=1, keepdims=True) / denom)

    @pl.when(i == 0)
    def _init():
        mon_ref[0, 0] = 0.0

    mon_ref[0, 0] = mon_ref[0, 0] + part

    @pl.when(i == n_blocks - 1)
    def _final():
        mon_ref[0, 0] = mon_ref[0, 0] / n_rows


def kernel(global_features, W1, b1, W2, b2, ema_load):
    n, d = global_features.shape
    h_dim = W1.shape[0]
    e_dim = W2.shape[0]
    bn = 1024
    n_blocks = n // bn

    b1r = b1.reshape(1, h_dim)
    b2r = b2.reshape(1, e_dim)

    x_spec = pl.BlockSpec((bn, d), lambda i: (i, 0))
    w1_spec = pl.BlockSpec((h_dim, d), lambda i: (0, 0))
    b1_spec = pl.BlockSpec((1, h_dim), lambda i: (0, 0))
    w2_spec = pl.BlockSpec((e_dim, h_dim), lambda i: (0, 0))
    b2_spec = pl.BlockSpec((1, e_dim), lambda i: (0, 0))
    smem_spec = pl.BlockSpec((1, 1), lambda i: (0, 0),
                             memory_space=pltpu.SMEM)

    fast = functools.partial(_fast_body, n_experts=e_dim)
    coeffs_f, mon_f, cv, cnt = pl.pallas_call(
        fast,
        grid=(n_blocks,),
        in_specs=[x_spec, w1_spec, b1_spec, w2_spec,
                  pl.BlockSpec((e_dim, 1), lambda i: (0, 0)),
                  pl.BlockSpec((1, e_dim), lambda i: (0, 0))],
        out_specs=[pl.BlockSpec((bn, e_dim), lambda i: (i, 0)),
                   smem_spec, smem_spec, smem_spec],
        out_shape=[
            jax.ShapeDtypeStruct((n, e_dim), jnp.float32),
            jax.ShapeDtypeStruct((1, 1), jnp.float32),
            jax.ShapeDtypeStruct((1, 1), jnp.float32),
            jax.ShapeDtypeStruct((1, 1), jnp.float32),
        ],
    )(global_features, W1, b1r, W2, b2.reshape(e_dim, 1),
      ema_load.reshape(1, e_dim))

    def _fast_result():
        return coeffs_f, mon_f[0, 0] / n

    def _exact_result():
        exact = functools.partial(_exact_body, n_blocks=n_blocks,
                                  n_rows=n, n_experts=e_dim)
        coeffs_e, mon_e = pl.pallas_call(
            exact,
            grid=(n_blocks,),
            in_specs=[x_spec, w1_spec, b1_spec, w2_spec, b2_spec],
            out_specs=[pl.BlockSpec((bn, e_dim), lambda i: (i, 0)),
                       smem_spec],
            out_shape=[
                jax.ShapeDtypeStruct((n, e_dim), jnp.float32),
                jax.ShapeDtypeStruct((1, 1), jnp.float32),
            ],
        )(global_features, W1, b1r, W2, b2r)
        return coeffs_e, mon_e[0, 0]

    coeffs, mon = jax.lax.cond(cnt[0, 0] == float(_K * n),
                               _fast_result, _exact_result)
    return coeffs, mon, cv[0, 0]
